# baseline (speedup 1.0000x reference)
import functools

import jax
import jax.numpy as jnp
from jax import lax
from jax.experimental import pallas as pl
from jax.experimental.pallas import tpu as pltpu


def _ln_stream_kernel(x_hbm, w_ref, b_ref, o_hbm, buf, in_sem, out_sem, *,
                      eps, inv_c, chunk, nchunks):
    i = pl.program_id(0)
    base = i * (chunk * nchunks)

    def in_copy(k):
        return pltpu.make_async_copy(
            x_hbm.at[pl.ds(base + k * chunk, chunk), :],
            buf.at[k],
            in_sem.at[k],
        )

    def out_copy(k):
        return pltpu.make_async_copy(
            buf.at[k],
            o_hbm.at[pl.ds(base + k * chunk, chunk), :],
            out_sem.at[k],
        )

    for k in range(nchunks):
        in_copy(k).start()
    w = w_ref[...]
    b = b_ref[...]
    for k in range(nchunks):
        in_copy(k).wait()
        x = buf[k]
        s = jnp.sum(x, axis=-1, keepdims=True)
        sq = jnp.sum(x * x, axis=-1, keepdims=True)
        mu = s * inv_c
        var = sq * inv_c - mu * mu
        inv = lax.rsqrt(var + eps)
        buf[k] = (x - mu) * inv * w + b
        out_copy(k).start()
    for k in range(nchunks):
        out_copy(k).wait()


def kernel(x, weight, bias, *, eps=1e-6):
    c = x.shape[-1]
    lead = x.shape[:-1]
    x2d = x.reshape(-1, c)
    rows = x2d.shape[0]

    ncores = 2
    nchunks = 8
    chunk = rows // (ncores * nchunks)

    kernel_fn = functools.partial(
        _ln_stream_kernel, eps=eps, inv_c=1.0 / c, chunk=chunk, nchunks=nchunks)
    y2d = pl.pallas_call(
        kernel_fn,
        out_shape=jax.ShapeDtypeStruct((rows, c), x.dtype),
        grid=(ncores,),
        in_specs=[
            pl.BlockSpec(memory_space=pl.ANY),
            pl.BlockSpec((1, c), lambda i: (0, 0)),
            pl.BlockSpec((1, c), lambda i: (0, 0)),
        ],
        out_specs=pl.BlockSpec(memory_space=pl.ANY),
        scratch_shapes=[
            pltpu.VMEM((nchunks, chunk, c), x.dtype),
            pltpu.SemaphoreType.DMA((nchunks,)),
            pltpu.SemaphoreType.DMA((nchunks,)),
        ],
        compiler_params=pltpu.CompilerParams(
            dimension_semantics=("parallel",),
            vmem_limit_bytes=48 * 1024 * 1024,
        ),
    )(x2d, weight.reshape(1, c), bias.reshape(1, c))
    return y2d.reshape(*lead, c)


# speedup vs baseline: 1.0247x; 1.0247x over previous
import functools

import jax
import jax.numpy as jnp
from jax import lax
from jax.experimental import pallas as pl
from jax.experimental.pallas import tpu as pltpu


def _ln_stream_kernel(x_hbm, w_ref, b_ref, o_hbm, buf, in_sem, out_sem, *,
                      eps, inv_c, chunk, nchunks):
    i = pl.program_id(0)
    base = i * (chunk * nchunks)

    def in_copy(k):
        return pltpu.make_async_copy(
            x_hbm.at[pl.ds(base + k * chunk, chunk), :],
            buf.at[k],
            in_sem.at[k],
        )

    def out_copy(k):
        return pltpu.make_async_copy(
            buf.at[k],
            o_hbm.at[pl.ds(base + k * chunk, chunk), :],
            out_sem.at[k],
        )

    for k in range(nchunks):
        in_copy(k).start()
    w = w_ref[...]
    b = b_ref[...]
    for k in range(nchunks):
        in_copy(k).wait()
        x = buf[k]
        s = jnp.sum(x, axis=-1, keepdims=True)
        sq = jnp.sum(x * x, axis=-1, keepdims=True)
        mu = s * inv_c
        var = sq * inv_c - mu * mu
        inv = lax.rsqrt(var + eps)
        buf[k] = (x - mu) * inv * w + b
        out_copy(k).start()
    for k in range(nchunks):
        out_copy(k).wait()


def kernel(x, weight, bias, *, eps=1e-6):
    c = x.shape[-1]
    lead = x.shape[:-1]
    x2d = x.reshape(-1, c)
    rows = x2d.shape[0]

    ncores = 2
    nchunks = 4
    chunk = rows // (ncores * nchunks)

    kernel_fn = functools.partial(
        _ln_stream_kernel, eps=eps, inv_c=1.0 / c, chunk=chunk, nchunks=nchunks)
    y2d = pl.pallas_call(
        kernel_fn,
        out_shape=jax.ShapeDtypeStruct((rows, c), x.dtype),
        grid=(ncores,),
        in_specs=[
            pl.BlockSpec(memory_space=pl.ANY),
            pl.BlockSpec((1, c), lambda i: (0, 0)),
            pl.BlockSpec((1, c), lambda i: (0, 0)),
        ],
        out_specs=pl.BlockSpec(memory_space=pl.ANY),
        scratch_shapes=[
            pltpu.VMEM((nchunks, chunk, c), x.dtype),
            pltpu.SemaphoreType.DMA((nchunks,)),
            pltpu.SemaphoreType.DMA((nchunks,)),
        ],
        compiler_params=pltpu.CompilerParams(
            dimension_semantics=("parallel",),
            vmem_limit_bytes=48 * 1024 * 1024,
        ),
    )(x2d, weight.reshape(1, c), bias.reshape(1, c))
    return y2d.reshape(*lead, c)
